# baseline (speedup 1.0000x reference)
import functools

import jax
import jax.numpy as jnp
from jax import lax
from jax.experimental import pallas as pl
from jax.experimental.pallas import tpu as pltpu
from jax.experimental.pallas import tpu_sc as plsc

NN = 10000
EE = 320000
DD = 128
NPAD = 10240
HALF = NPAD // 2
HB = HALF + 128
NW = 32
CHUNK = 64
EPW = 10240
EPAD = EPW * NW
GCAP = EPW + CHUNK
NBUF = 3
BLK = 1024
GRID = NPAD // BLK

_mesh = plsc.VectorSubcoreMesh(core_axis_name="c", subcore_axis_name="s")
_sc_params = pltpu.CompilerParams(needs_layout_passes=False)


@functools.partial(
    pl.kernel,
    out_type=jax.ShapeDtypeStruct((64, NPAD), jnp.float32),
    mesh=_mesh,
    scratch_types=[
        pltpu.VMEM((EPW,), jnp.int32),
        pltpu.VMEM((NPAD,), jnp.float32),
    ],
    compiler_params=_sc_params,
)
def _deg_kernel(src_hbm, dst_hbm, out_hbm, idxbuf, hist):
    c = lax.axis_index("c")
    s = lax.axis_index("s")
    wid = s * 2 + c
    zeros16 = jnp.zeros((16,), jnp.float32)
    ones16 = jnp.ones((16,), jnp.float32)
    for half, edges in ((0, src_hbm), (1, dst_hbm)):
        def zbody(i, _):
            hist[pl.ds(i * 16, 16)] = zeros16
            return 0
        lax.fori_loop(0, NPAD // 16, zbody, 0)
        pltpu.sync_copy(edges.at[pl.ds(wid * EPW, EPW)], idxbuf)

        def body(i, _):
            idx = idxbuf[pl.ds(i * 16, 16)]
            plsc.addupdate_scatter(hist, [idx], ones16)
            return 0
        lax.fori_loop(0, EPW // 16, body, 0)
        pltpu.sync_copy(hist, out_hbm.at[half * 32 + wid])


@functools.partial(
    pl.kernel,
    out_type=(
        jax.ShapeDtypeStruct((4, NW, GCAP), jnp.int32),
        jax.ShapeDtypeStruct((4, NW, GCAP), jnp.int32),
        jax.ShapeDtypeStruct((NW, 16), jnp.int32),
    ),
    mesh=_mesh,
    scratch_types=[
        pltpu.VMEM((EPW,), jnp.int32),
        pltpu.VMEM((EPW,), jnp.int32),
        [pltpu.VMEM((GCAP,), jnp.int32) for _ in range(4)],
        [pltpu.VMEM((GCAP,), jnp.int32) for _ in range(4)],
        pltpu.VMEM((16,), jnp.int32),
    ],
    compiler_params=_sc_params,
)
def _part_kernel(src_hbm, dst_hbm, sg_out, dg_out, cnt_out,
                 sv, dv, sgb, dgb, cnt_v):
    c = lax.axis_index("c")
    s = lax.axis_index("s")
    wid = s * 2 + c
    pltpu.sync_copy(src_hbm.at[pl.ds(wid * EPW, EPW)], sv)
    pltpu.sync_copy(dst_hbm.at[pl.ds(wid * EPW, EPW)], dv)

    def body(i, cnts):
        s16 = sv[pl.ds(i * 16, 16)]
        d16 = dv[pl.ds(i * 16, 16)]
        new = []
        for g in range(4):
            dh, sh = g // 2, g % 2
            m_d = (d16 < HALF) if dh == 0 else (d16 >= HALF)
            m_s = (s16 < HALF) if sh == 0 else (s16 >= HALF)
            m = m_d & m_s
            cg = cnts[g]
            plsc.store_compressed(sgb[g].at[pl.ds(cg, 16)], s16 - sh * HALF, mask=m)
            plsc.store_compressed(dgb[g].at[pl.ds(cg, 16)], d16 - dh * HALF, mask=m)
            new.append(cg + jnp.sum(m.astype(jnp.int32)))
        return tuple(new)
    cnts = lax.fori_loop(0, EPW // 16, body, (0, 0, 0, 0))

    pad16 = jnp.full((16,), HALF, jnp.int32)
    for g in range(4):
        cg = cnts[g]
        for k in range(CHUNK // 16):
            sgb[g][pl.ds(cg + k * 16, 16)] = pad16
            dgb[g][pl.ds(cg + k * 16, 16)] = pad16
    iota16 = lax.iota(jnp.int32, 16)
    cvec = jnp.zeros((16,), jnp.int32)
    for g in range(4):
        padded = ((cnts[g] + CHUNK - 1) // CHUNK) * CHUNK
        cvec = cvec + jnp.where(iota16 == g, padded, 0)
    cnt_v[pl.ds(0, 16)] = cvec
    for g in range(4):
        pltpu.sync_copy(sgb[g], sg_out.at[g, wid])
        pltpu.sync_copy(dgb[g], dg_out.at[g, wid])
    pltpu.sync_copy(cnt_v, cnt_out.at[wid])


@functools.partial(
    pl.kernel,
    out_type=jax.ShapeDtypeStruct((2, NPAD, DD), jnp.float32),
    mesh=_mesh,
    scratch_types=[
        pltpu.VMEM((GCAP,), jnp.int32),
        pltpu.VMEM((GCAP,), jnp.int32),
        pltpu.VMEM((CHUNK,), jnp.int32),
        pltpu.VMEM((16,), jnp.int32),
        pltpu.VMEM((NBUF, CHUNK, DD), jnp.float32),
        pltpu.VMEM_SHARED((HB, DD), jnp.float32),
        pltpu.VMEM_SHARED((HB, DD), jnp.float32),
        pltpu.SemaphoreType.DMA((NBUF,)),
    ],
    compiler_params=_sc_params,
)
def _agg_kernel(hp_hbm, sg_hbm, dg_hbm, cnt_hbm, out_hbm,
                sv, dv, idxd, cnt_v, rows, hp_buf, acc, sems):
    c = lax.axis_index("c")
    s = lax.axis_index("s")
    wid = s * 2 + c
    zeros16 = jnp.zeros((16,), jnp.float32)

    pltpu.sync_copy(cnt_hbm.at[wid], cnt_v)
    cvec = cnt_v[pl.ds(0, 16)]

    def zbody(i, _):
        for k in range(DD // 16):
            rows[0, i, pl.ds(k * 16, 16)] = zeros16
        return 0
    lax.fori_loop(0, CHUNK, zbody, 0)
    @pl.when(s == 0)
    def _():
        pltpu.sync_copy(rows.at[0], hp_buf.at[pl.ds(HALF, CHUNK)])
        pltpu.sync_copy(rows.at[0], hp_buf.at[pl.ds(HALF + CHUNK, CHUNK)])

    for dh in range(2):
        if dh > 0:
            lax.fori_loop(0, CHUNK, zbody, 0)
        for t in range(5):
            pltpu.sync_copy(rows.at[0], acc.at[pl.ds(s * 328 + t * CHUNK, CHUNK)])
        pltpu.sync_copy(rows.at[0].at[pl.ds(0, 8)], acc.at[pl.ds(s * 328 + 320, 8)])
        plsc.subcore_barrier()

        for sh in range(2):
            g = dh * 2 + sh
            pltpu.sync_copy(
                hp_hbm.at[pl.ds(sh * HALF + s * 320, 320)],
                hp_buf.at[pl.ds(s * 320, 320)],
            )
            plsc.subcore_barrier()
            pltpu.sync_copy(sg_hbm.at[g, wid], sv)
            pltpu.sync_copy(dg_hbm.at[g, wid], dv)
            trips = cvec[g] // CHUNK
            for b in range(NBUF):
                @pl.when(b < trips)
                def _():
                    pltpu.async_copy(
                        hp_buf.at[sv.at[pl.ds(b * CHUNK, CHUNK)]],
                        rows.at[b], sems.at[b])

            def body(t, _):
                bi = lax.rem(t, NBUF)
                pltpu.make_async_copy(
                    hp_buf.at[sv.at[pl.ds(t * CHUNK, CHUNK)]],
                    rows.at[bi], sems.at[bi],
                ).wait()
                for k in range(CHUNK // 16):
                    idxd[pl.ds(k * 16, 16)] = dv[pl.ds(t * CHUNK + k * 16, 16)]
                pltpu.sync_copy(rows.at[bi], acc.at[idxd], add=True)

                @pl.when(t + NBUF < trips)
                def _():
                    pltpu.async_copy(
                        hp_buf.at[sv.at[pl.ds((t + NBUF) * CHUNK, CHUNK)]],
                        rows.at[bi], sems.at[bi])
                return 0
            lax.fori_loop(0, trips, body, 0)
            plsc.subcore_barrier()

        pltpu.sync_copy(
            acc.at[pl.ds(s * 320, 320)],
            out_hbm.at[c].at[pl.ds(dh * HALF + s * 320, 320)],
        )
        plsc.subcore_barrier()


def _norms_body(h_ref, out_ref):
    dego = jnp.sum(h_ref[0:32, :], axis=0)
    degi = jnp.sum(h_ref[32:64, :], axis=0)
    ono = jnp.where(dego > 0, lax.rsqrt(jnp.maximum(dego, 1.0)), 0.0)
    oni = jnp.where(degi > 0, lax.rsqrt(jnp.maximum(degi, 1.0)), 0.0)
    out_ref[...] = jnp.stack([ono, oni])


_norms = pl.pallas_call(
    _norms_body,
    out_shape=jax.ShapeDtypeStruct((2, NPAD), jnp.float32),
)


def _l0_body(x_ref, norms_ref, w_ref, out_ref):
    h = x_ref[...] * norms_ref[0, :][:, None]
    out_ref[...] = jnp.dot(h, w_ref[...], preferred_element_type=jnp.float32)


_l0 = pl.pallas_call(
    _l0_body,
    grid=(GRID,),
    in_specs=[
        pl.BlockSpec((BLK, DD), lambda i: (i, 0)),
        pl.BlockSpec((2, BLK), lambda i: (0, i)),
        pl.BlockSpec((DD, DD), lambda i: (0, 0)),
    ],
    out_specs=pl.BlockSpec((BLK, DD), lambda i: (i, 0)),
    out_shape=jax.ShapeDtypeStruct((NPAD, DD), jnp.float32),
)


def _mid_body(parts_ref, norms_ref, b_ref, w_ref, out_ref):
    i = pl.program_id(0)
    agg = parts_ref[0] + parts_ref[1]
    h = jnp.maximum(agg * norms_ref[1, :][:, None] + b_ref[...], 0.0)
    r = i * BLK + lax.broadcasted_iota(jnp.int32, (BLK, 1), 0)
    h = jnp.where(r < NN, h * norms_ref[0, :][:, None], 0.0)
    out_ref[...] = jnp.dot(h, w_ref[...], preferred_element_type=jnp.float32)


_mid = pl.pallas_call(
    _mid_body,
    grid=(GRID,),
    in_specs=[
        pl.BlockSpec((2, BLK, DD), lambda i: (0, i, 0)),
        pl.BlockSpec((2, BLK), lambda i: (0, i)),
        pl.BlockSpec((1, DD), lambda i: (0, 0)),
        pl.BlockSpec((DD, DD), lambda i: (0, 0)),
    ],
    out_specs=pl.BlockSpec((BLK, DD), lambda i: (i, 0)),
    out_shape=jax.ShapeDtypeStruct((NPAD, DD), jnp.float32),
)


def _fin_body(parts_ref, norms_ref, b_ref, wc_ref, bc_ref, out_ref, acc_ref):
    i = pl.program_id(0)

    @pl.when(i == 0)
    def _():
        acc_ref[...] = jnp.zeros_like(acc_ref)

    agg = parts_ref[0] + parts_ref[1]
    h = jnp.maximum(agg * norms_ref[1, :][:, None] + b_ref[...], 0.0)
    r = i * BLK + lax.broadcasted_iota(jnp.int32, (BLK, 1), 0)
    h = jnp.where(r < NN, h, 0.0)
    acc_ref[...] += jnp.sum(h, axis=0, keepdims=True)

    @pl.when(i == GRID - 1)
    def _():
        hg = acc_ref[...] * (1.0 / NN)
        out_ref[...] = (
            jnp.dot(hg, wc_ref[...], preferred_element_type=jnp.float32)
            + bc_ref[...]
        )


_fin = pl.pallas_call(
    _fin_body,
    grid=(GRID,),
    in_specs=[
        pl.BlockSpec((2, BLK, DD), lambda i: (0, i, 0)),
        pl.BlockSpec((2, BLK), lambda i: (0, i)),
        pl.BlockSpec((1, DD), lambda i: (0, 0)),
        pl.BlockSpec((DD, DD), lambda i: (0, 0)),
        pl.BlockSpec((1, DD), lambda i: (0, 0)),
    ],
    out_specs=pl.BlockSpec((1, DD), lambda i: (0, 0)),
    out_shape=jax.ShapeDtypeStruct((1, DD), jnp.float32),
    scratch_shapes=[pltpu.VMEM((1, DD), jnp.float32)],
)


@jax.jit
def _run(x, src, dst, W0, b0, W1, b1, W2, b2, Wc, bc):
    pad = EPAD - EE
    src_p = jnp.concatenate([src, jnp.full((pad,), NN, jnp.int32)])
    dst_p = jnp.concatenate([dst, jnp.full((pad,), NN, jnp.int32)])
    x_p = jnp.concatenate([x, jnp.zeros((NPAD - NN, DD), jnp.float32)], axis=0)
    wc_p = jnp.pad(Wc, ((0, 0), (0, DD - Wc.shape[1])))
    bc_p = jnp.pad(bc, (0, DD - bc.shape[0]))[None, :]

    hists = _deg_kernel(src_p, dst_p)
    sg, dg, cnts = _part_kernel(src_p, dst_p)
    norms = _norms(hists)
    hp = _l0(x_p, norms, W0)
    parts = _agg_kernel(hp, sg, dg, cnts)
    hp = _mid(parts, norms, b0[None, :], W1)
    parts = _agg_kernel(hp, sg, dg, cnts)
    hp = _mid(parts, norms, b1[None, :], W2)
    parts = _agg_kernel(hp, sg, dg, cnts)
    out = _fin(parts, norms, b2[None, :], wc_p, bc_p)
    return out[0, : Wc.shape[1]]


def kernel(x, edge_index, W0, b0, W1, b1, W2, b2, Wc, bc):
    src = edge_index[0].astype(jnp.int32)
    dst = edge_index[1].astype(jnp.int32)
    return _run(x, src, dst, W0, b0, W1, b1, W2, b2, Wc, bc)
